# baseline (speedup 1.0000x reference)
import functools

import jax
import jax.numpy as jnp
from jax import lax
from jax.experimental import pallas as pl
from jax.experimental.pallas import tpu as pltpu
from jax.experimental.pallas import tpu_sc as plsc

N = 10000
NP = 10240
D = 256
H1 = 128
E = 160000
EP = 163840
NC = 2
NT = 16
NW = NC * NT
WIN = 32
NWIN = EP // (NW * WIN)
DEPTH = 8
EPT = EP // NW
RPT = NP // NT
VL = 16

_mesh = plsc.VectorSubcoreMesh(core_axis_name="c", subcore_axis_name="s")


def _zero_1d(ref, nwords):
    def body(i, _):
        ref[pl.ds(i * VL, VL)] = jnp.zeros((VL,), jnp.float32)
        return 0
    lax.fori_loop(0, nwords // VL, body, 0)


def _zero_2d(ref, nrows, ncols):
    def body(i, _):
        for k in range(ncols // VL):
            ref[i, pl.ds(k * VL, VL)] = jnp.zeros((VL,), jnp.float32)
        return 0
    lax.fori_loop(0, nrows, body, 0)


@functools.partial(
    pl.kernel,
    out_type=jax.ShapeDtypeStruct((NW, NP), jnp.float32),
    mesh=_mesh,
    scratch_types=[
        pltpu.MemorySpace.VMEM((EPT,), jnp.int32),
        pltpu.MemorySpace.VMEM((NP,), jnp.float32),
    ],
    compiler_params=pltpu.CompilerParams(needs_layout_passes=False),
)
def _deg_kernel(dst_hbm, out, dst_v, acc_v):
    c = lax.axis_index("c")
    s = lax.axis_index("s")
    w = c * NT + s

    _zero_1d(acc_v, NP)
    pltpu.sync_copy(dst_hbm.at[pl.ds(w * EPT, EPT)], dst_v)

    ones = jnp.ones((VL,), jnp.float32)

    def body(i, _):
        idx = dst_v[pl.ds(i * VL, VL)]
        plsc.addupdate_scatter(acc_v, [idx], ones)
        return 0
    lax.fori_loop(0, EPT // VL, body, 0)

    pltpu.sync_copy(acc_v, out.at[w])


@functools.partial(
    pl.kernel,
    out_type=[jax.ShapeDtypeStruct((NP, H1), jnp.float32) for _ in range(NC)],
    mesh=_mesh,
    scratch_types=[
        pltpu.MemorySpace.VMEM((NWIN // 2, 4 * WIN), jnp.int32),
        [pltpu.MemorySpace.VMEM((WIN, H1), jnp.float32) for _ in range(DEPTH)],
        pltpu.MemorySpace.VMEM_SHARED((NP, H1), jnp.float32),
        [pltpu.SemaphoreType.DMA for _ in range(DEPTH)],
    ],
)
def _agg_kernel(g_hbm, sd_hbm, out0, out1, sd_v, gbs, agg_s, sems):
    c = lax.axis_index("c")
    s = lax.axis_index("s")
    w = c * NT + s

    _zero_2d(gbs[0], WIN, H1)
    for k in range(RPT // WIN):
        pltpu.sync_copy(gbs[0], agg_s.at[pl.ds(s * RPT + k * WIN, WIN)])
    pltpu.sync_copy(sd_hbm.at[pl.ds(w * (NWIN // 2), NWIN // 2)], sd_v)
    plsc.subcore_barrier()

    def _src(row, k):
        return sd_v.at[row, pl.ds((k % 2) * 2 * WIN, WIN)]

    def _dst(row, k):
        return sd_v.at[row, pl.ds((k % 2) * 2 * WIN + WIN, WIN)]

    for k in range(DEPTH):
        pltpu.async_copy(g_hbm.at[_src(k // 2, k)], gbs[k], sems[k])

    def step(u, _):
        for k in range(DEPTH):
            row = (DEPTH // 2) * u + k // 2
            pltpu.make_async_copy(
                g_hbm.at[_src(row, k)], gbs[k], sems[k]).wait()
            pltpu.sync_copy(gbs[k], agg_s.at[_dst(row, k)], add=True)

            @pl.when((DEPTH * u + k) + DEPTH < NWIN)
            def _():
                pltpu.async_copy(
                    g_hbm.at[_src(row + DEPTH // 2, k)], gbs[k], sems[k])
        return 0
    lax.fori_loop(0, NWIN // DEPTH, step, 0)
    plsc.subcore_barrier()

    @pl.when(c == 0)
    def _():
        pltpu.sync_copy(agg_s.at[pl.ds(s * RPT, RPT)], out0.at[pl.ds(s * RPT, RPT)])

    @pl.when(c == 1)
    def _():
        pltpu.sync_copy(agg_s.at[pl.ds(s * RPT, RPT)], out1.at[pl.ds(s * RPT, RPT)])


@functools.partial(
    pl.kernel,
    out_type=jax.ShapeDtypeStruct((NW, NP), jnp.float32),
    mesh=_mesh,
    scratch_types=[
        pltpu.MemorySpace.VMEM((EPT,), jnp.int32),
        pltpu.MemorySpace.VMEM((EPT,), jnp.int32),
        pltpu.MemorySpace.VMEM((NP,), jnp.float32),
        pltpu.MemorySpace.VMEM((NP,), jnp.float32),
    ],
    compiler_params=pltpu.CompilerParams(needs_layout_passes=False),
)
def _scal_kernel(gz_hbm, src_hbm, dst_hbm, out, src_v, dst_v, gz_v, acc_v):
    c = lax.axis_index("c")
    s = lax.axis_index("s")
    w = c * NT + s

    _zero_1d(acc_v, NP)
    pltpu.sync_copy(gz_hbm, gz_v)
    pltpu.sync_copy(src_hbm.at[pl.ds(w * EPT, EPT)], src_v)
    pltpu.sync_copy(dst_hbm.at[pl.ds(w * EPT, EPT)], dst_v)

    def body(i, _):
        si = src_v[pl.ds(i * VL, VL)]
        di = dst_v[pl.ds(i * VL, VL)]
        v = plsc.load_gather(gz_v, [si])
        plsc.addupdate_scatter(acc_v, [di], v)
        return 0
    lax.fori_loop(0, EPT // VL, body, 0)

    pltpu.sync_copy(acc_v, out.at[w])


BR = 2048


def _tc1_body(x_ref, w1_ref, dp_ref, g_ref, dinv_ref):
    deg = jnp.sum(dp_ref[...], axis=0) + 1.0
    dinv = lax.rsqrt(jnp.maximum(deg, 1e-12))
    h = jnp.dot(x_ref[...], w1_ref[...], preferred_element_type=jnp.float32)
    g_ref[...] = h * dinv[:, None]
    dinv_ref[...] = dinv


_tc1 = pl.pallas_call(
    _tc1_body,
    grid=(NP // BR,),
    in_specs=[
        pl.BlockSpec((BR, D), lambda i: (i, 0)),
        pl.BlockSpec((D, H1), lambda i: (0, 0)),
        pl.BlockSpec((NW, BR), lambda i: (0, i)),
    ],
    out_specs=[
        pl.BlockSpec((BR, H1), lambda i: (i, 0)),
        pl.BlockSpec((BR,), lambda i: (i,)),
    ],
    out_shape=[
        jax.ShapeDtypeStruct((NP, H1), jnp.float32),
        jax.ShapeDtypeStruct((NP,), jnp.float32),
    ],
)


def _tc2_body(a0_ref, a1_ref, g_ref, dinv_ref, b1_ref, w2_ref, wl_ref, gz_ref):
    dinv = dinv_ref[...]
    pre = (a0_ref[...] + a1_ref[...] + g_ref[...]) * dinv[:, None] + b1_ref[...]
    out1 = jnp.maximum(pre, 0.0)
    cvec = jnp.dot(w2_ref[...], wl_ref[...], preferred_element_type=jnp.float32)
    z = jnp.dot(out1, cvec, preferred_element_type=jnp.float32)
    gz_ref[...] = z[:, 0] * dinv


_tc2 = pl.pallas_call(
    _tc2_body,
    grid=(NP // BR,),
    in_specs=[
        pl.BlockSpec((BR, H1), lambda i: (i, 0)),
        pl.BlockSpec((BR, H1), lambda i: (i, 0)),
        pl.BlockSpec((BR, H1), lambda i: (i, 0)),
        pl.BlockSpec((BR,), lambda i: (i,)),
        pl.BlockSpec((1, H1), lambda i: (0, 0)),
        pl.BlockSpec((H1, 64), lambda i: (0, 0)),
        pl.BlockSpec((64, 1), lambda i: (0, 0)),
    ],
    out_specs=pl.BlockSpec((BR,), lambda i: (i,)),
    out_shape=jax.ShapeDtypeStruct((NP,), jnp.float32),
)


def _tc3_body(zp_ref, gz_ref, dinv_ref, b2_ref, wlf_ref, bl_ref, o_ref):
    z = jnp.sum(zp_ref[...], axis=0)
    const = jnp.sum(b2_ref[...] * wlf_ref[...]) + bl_ref[0]
    o_ref[...] = dinv_ref[...] * (z + gz_ref[...]) + const


_tc3 = pl.pallas_call(
    _tc3_body,
    grid=(1,),
    in_specs=[
        pl.BlockSpec((NW, NP), lambda i: (0, 0)),
        pl.BlockSpec((NP,), lambda i: (0,)),
        pl.BlockSpec((NP,), lambda i: (0,)),
        pl.BlockSpec((64,), lambda i: (0,)),
        pl.BlockSpec((64,), lambda i: (0,)),
        pl.BlockSpec((1,), lambda i: (0,)),
    ],
    out_specs=pl.BlockSpec((NP,), lambda i: (0,)),
    out_shape=jax.ShapeDtypeStruct((NP,), jnp.float32),
)


def kernel(x, edge_index, W1, b1, W2, b2, Wl, bl):
    src = edge_index[0].astype(jnp.int32)
    dst = edge_index[1].astype(jnp.int32)
    pad_pos = N + (jnp.arange(EP - E, dtype=jnp.int32) % (NP - N))
    src_f = jnp.concatenate([src, pad_pos])
    dst_f = jnp.concatenate([dst, pad_pos])
    sd2d = jnp.stack(
        [src_f.reshape(EP // WIN, WIN), dst_f.reshape(EP // WIN, WIN)],
        axis=1).reshape(EP // (2 * WIN), 4 * WIN)
    xp = jnp.pad(x, ((0, NP - N), (0, 0)))

    dparts = _deg_kernel(dst_f)
    g, dinv = _tc1(xp, W1, dparts)
    a0, a1 = _agg_kernel(g, sd2d)
    gz = _tc2(a0, a1, g, dinv, b1.reshape(1, H1), W2, Wl)
    zparts = _scal_kernel(gz, src_f, dst_f)
    out = _tc3(zparts, gz, dinv, b2, Wl[:, 0], bl)
    return out[:N]


# speedup vs baseline: 1.2326x; 1.2326x over previous
import functools

import jax
import jax.numpy as jnp
from jax import lax
from jax.experimental import pallas as pl
from jax.experimental.pallas import tpu as pltpu
from jax.experimental.pallas import tpu_sc as plsc

N = 10000
NP = 10240
D = 256
H1 = 128
E = 160000
EP = 163840
NC = 2
NT = 16
NW = NC * NT
WIN = 64
NWIN = EP // (NW * WIN)
DEPTH = 4
EPT = EP // NW
RPT = NP // NT
VL = 16

_mesh = plsc.VectorSubcoreMesh(core_axis_name="c", subcore_axis_name="s")


def _zero_1d(ref, nwords):
    def body(i, _):
        ref[pl.ds(i * VL, VL)] = jnp.zeros((VL,), jnp.float32)
        return 0
    lax.fori_loop(0, nwords // VL, body, 0)


def _zero_2d(ref, nrows, ncols):
    def body(i, _):
        for k in range(ncols // VL):
            ref[i, pl.ds(k * VL, VL)] = jnp.zeros((VL,), jnp.float32)
        return 0
    lax.fori_loop(0, nrows, body, 0)


@functools.partial(
    pl.kernel,
    out_type=jax.ShapeDtypeStruct((NW, NP), jnp.float32),
    mesh=_mesh,
    scratch_types=[
        pltpu.MemorySpace.VMEM((EPT,), jnp.int32),
        pltpu.MemorySpace.VMEM((NP,), jnp.float32),
    ],
    compiler_params=pltpu.CompilerParams(needs_layout_passes=False),
)
def _deg_kernel(dst_hbm, out, dst_v, acc_v):
    c = lax.axis_index("c")
    s = lax.axis_index("s")
    w = c * NT + s

    _zero_1d(acc_v, NP)
    pltpu.sync_copy(dst_hbm.at[pl.ds(w * EPT, EPT)], dst_v)

    ones = jnp.ones((VL,), jnp.float32)

    def body(i, _):
        idx = dst_v[pl.ds(i * VL, VL)]
        plsc.addupdate_scatter(acc_v, [idx], ones)
        return 0
    lax.fori_loop(0, EPT // VL, body, 0)

    pltpu.sync_copy(acc_v, out.at[w])


@functools.partial(
    pl.kernel,
    out_type=[jax.ShapeDtypeStruct((NP, H1), jnp.float32) for _ in range(NC)],
    mesh=_mesh,
    scratch_types=[
        pltpu.MemorySpace.VMEM((NWIN, 2 * WIN), jnp.int32),
        [pltpu.MemorySpace.VMEM((WIN, H1), jnp.float32) for _ in range(DEPTH)],
        pltpu.MemorySpace.VMEM_SHARED((NP, H1), jnp.float32),
        [pltpu.SemaphoreType.DMA for _ in range(DEPTH)],
    ],
)
def _agg_kernel(g_hbm, sd_hbm, out0, out1, sd_v, gbs, agg_s, sems):
    c = lax.axis_index("c")
    s = lax.axis_index("s")
    w = c * NT + s

    _zero_2d(gbs[0], WIN, H1)
    for k in range(RPT // WIN):
        pltpu.sync_copy(gbs[0], agg_s.at[pl.ds(s * RPT + k * WIN, WIN)])
    pltpu.sync_copy(sd_hbm.at[pl.ds(w * NWIN, NWIN)], sd_v)
    plsc.subcore_barrier()

    for k in range(DEPTH):
        pltpu.async_copy(
            g_hbm.at[sd_v.at[k, pl.ds(0, WIN)]], gbs[k], sems[k])

    def step(u, _):
        for k in range(DEPTH):
            t = DEPTH * u + k
            pltpu.make_async_copy(
                g_hbm.at[sd_v.at[t, pl.ds(0, WIN)]], gbs[k], sems[k]).wait()
            pltpu.sync_copy(
                gbs[k], agg_s.at[sd_v.at[t, pl.ds(WIN, WIN)]], add=True)

            @pl.when(t + DEPTH < NWIN)
            def _():
                pltpu.async_copy(
                    g_hbm.at[sd_v.at[t + DEPTH, pl.ds(0, WIN)]],
                    gbs[k], sems[k])
        return 0
    lax.fori_loop(0, NWIN // DEPTH, step, 0)
    for k in range(NWIN % DEPTH):
        t = (NWIN // DEPTH) * DEPTH + k
        pltpu.make_async_copy(
            g_hbm.at[sd_v.at[t, pl.ds(0, WIN)]], gbs[k], sems[k]).wait()
        pltpu.sync_copy(
            gbs[k], agg_s.at[sd_v.at[t, pl.ds(WIN, WIN)]], add=True)
    plsc.subcore_barrier()

    @pl.when(c == 0)
    def _():
        pltpu.sync_copy(agg_s.at[pl.ds(s * RPT, RPT)], out0.at[pl.ds(s * RPT, RPT)])

    @pl.when(c == 1)
    def _():
        pltpu.sync_copy(agg_s.at[pl.ds(s * RPT, RPT)], out1.at[pl.ds(s * RPT, RPT)])


@functools.partial(
    pl.kernel,
    out_type=jax.ShapeDtypeStruct((NW, NP), jnp.float32),
    mesh=_mesh,
    scratch_types=[
        pltpu.MemorySpace.VMEM((EPT,), jnp.int32),
        pltpu.MemorySpace.VMEM((EPT,), jnp.int32),
        pltpu.MemorySpace.VMEM((NP,), jnp.float32),
        pltpu.MemorySpace.VMEM((NP,), jnp.float32),
    ],
    compiler_params=pltpu.CompilerParams(needs_layout_passes=False),
)
def _scal_kernel(gz_hbm, src_hbm, dst_hbm, out, src_v, dst_v, gz_v, acc_v):
    c = lax.axis_index("c")
    s = lax.axis_index("s")
    w = c * NT + s

    _zero_1d(acc_v, NP)
    pltpu.sync_copy(gz_hbm, gz_v)
    pltpu.sync_copy(src_hbm.at[pl.ds(w * EPT, EPT)], src_v)
    pltpu.sync_copy(dst_hbm.at[pl.ds(w * EPT, EPT)], dst_v)

    def body(i, _):
        si = src_v[pl.ds(i * VL, VL)]
        di = dst_v[pl.ds(i * VL, VL)]
        v = plsc.load_gather(gz_v, [si])
        plsc.addupdate_scatter(acc_v, [di], v)
        return 0
    lax.fori_loop(0, EPT // VL, body, 0)

    pltpu.sync_copy(acc_v, out.at[w])


BR = 2048


def _tc1_body(x_ref, w1_ref, dp_ref, g_ref, dinv_ref):
    deg = jnp.sum(dp_ref[...], axis=0) + 1.0
    dinv = lax.rsqrt(jnp.maximum(deg, 1e-12))
    h = jnp.dot(x_ref[...], w1_ref[...], preferred_element_type=jnp.float32)
    g_ref[...] = h * dinv[:, None]
    dinv_ref[...] = dinv


_tc1 = pl.pallas_call(
    _tc1_body,
    grid=(NP // BR,),
    in_specs=[
        pl.BlockSpec((BR, D), lambda i: (i, 0)),
        pl.BlockSpec((D, H1), lambda i: (0, 0)),
        pl.BlockSpec((NW, BR), lambda i: (0, i)),
    ],
    out_specs=[
        pl.BlockSpec((BR, H1), lambda i: (i, 0)),
        pl.BlockSpec((BR,), lambda i: (i,)),
    ],
    out_shape=[
        jax.ShapeDtypeStruct((NP, H1), jnp.float32),
        jax.ShapeDtypeStruct((NP,), jnp.float32),
    ],
)


def _tc2_body(a0_ref, a1_ref, g_ref, dinv_ref, b1_ref, w2_ref, wl_ref, gz_ref):
    dinv = dinv_ref[...]
    pre = (a0_ref[...] + a1_ref[...] + g_ref[...]) * dinv[:, None] + b1_ref[...]
    out1 = jnp.maximum(pre, 0.0)
    cvec = jnp.dot(w2_ref[...], wl_ref[...], preferred_element_type=jnp.float32)
    z = jnp.dot(out1, cvec, preferred_element_type=jnp.float32)
    gz_ref[...] = z[:, 0] * dinv


_tc2 = pl.pallas_call(
    _tc2_body,
    grid=(NP // BR,),
    in_specs=[
        pl.BlockSpec((BR, H1), lambda i: (i, 0)),
        pl.BlockSpec((BR, H1), lambda i: (i, 0)),
        pl.BlockSpec((BR, H1), lambda i: (i, 0)),
        pl.BlockSpec((BR,), lambda i: (i,)),
        pl.BlockSpec((1, H1), lambda i: (0, 0)),
        pl.BlockSpec((H1, 64), lambda i: (0, 0)),
        pl.BlockSpec((64, 1), lambda i: (0, 0)),
    ],
    out_specs=pl.BlockSpec((BR,), lambda i: (i,)),
    out_shape=jax.ShapeDtypeStruct((NP,), jnp.float32),
)


def _tc3_body(zp_ref, gz_ref, dinv_ref, b2_ref, wlf_ref, bl_ref, o_ref):
    z = jnp.sum(zp_ref[...], axis=0)
    const = jnp.sum(b2_ref[...] * wlf_ref[...]) + bl_ref[0]
    o_ref[...] = dinv_ref[...] * (z + gz_ref[...]) + const


_tc3 = pl.pallas_call(
    _tc3_body,
    grid=(1,),
    in_specs=[
        pl.BlockSpec((NW, NP), lambda i: (0, 0)),
        pl.BlockSpec((NP,), lambda i: (0,)),
        pl.BlockSpec((NP,), lambda i: (0,)),
        pl.BlockSpec((64,), lambda i: (0,)),
        pl.BlockSpec((64,), lambda i: (0,)),
        pl.BlockSpec((1,), lambda i: (0,)),
    ],
    out_specs=pl.BlockSpec((NP,), lambda i: (0,)),
    out_shape=jax.ShapeDtypeStruct((NP,), jnp.float32),
)


def kernel(x, edge_index, W1, b1, W2, b2, Wl, bl):
    src = edge_index[0].astype(jnp.int32)
    dst = edge_index[1].astype(jnp.int32)
    pad_pos = N + (jnp.arange(EP - E, dtype=jnp.int32) % (NP - N))
    src_f = jnp.concatenate([src, pad_pos])
    dst_f = jnp.concatenate([dst, pad_pos])
    sd2d = jnp.concatenate(
        [src_f.reshape(EP // WIN, WIN), dst_f.reshape(EP // WIN, WIN)], axis=1)
    xp = jnp.pad(x, ((0, NP - N), (0, 0)))

    dparts = _deg_kernel(dst_f)
    g, dinv = _tc1(xp, W1, dparts)
    a0, a1 = _agg_kernel(g, sd2d)
    gz = _tc2(a0, a1, g, dinv, b1.reshape(1, H1), W2, Wl)
    zparts = _scal_kernel(gz, src_f, dst_f)
    out = _tc3(zparts, gz, dinv, b2, Wl[:, 0], bl)
    return out[:N]
